# baseline (speedup 1.0000x reference)
import functools

import jax
import jax.numpy as jnp
from jax import lax
from jax.experimental import pallas as pl
from jax.experimental.pallas import tpu as pltpu
from jax.experimental.pallas import tpu_sc as plsc

NC = 2
NS = 16
LANES = 16
CH = 128


def _sc_embed(tok, seg, table, comb, *, n_rows, d, n_pos):
    n_workers = NC * NS
    rows_per_worker = n_rows // n_workers
    n_chunks = rows_per_worker // CH
    n_comb = comb.shape[0]
    mesh = plsc.VectorSubcoreMesh(
        core_axis_name="c", subcore_axis_name="s",
        num_cores=NC, num_subcores=NS)

    @functools.partial(
        pl.kernel,
        out_type=jax.ShapeDtypeStruct((n_rows, d), jnp.float32),
        mesh=mesh,
        scratch_types=dict(
            comb_v=pltpu.VMEM((n_comb, d), jnp.float32),
            tok_v=[pltpu.VMEM((CH,), jnp.int32) for _ in range(2)],
            seg_v=[pltpu.VMEM((CH,), jnp.int32) for _ in range(2)],
            cidx_v=[pltpu.VMEM((CH,), jnp.int32) for _ in range(2)],
            rows_v=[pltpu.VMEM((CH, d), jnp.float32) for _ in range(2)],
            tsem=[pltpu.SemaphoreType.DMA for _ in range(2)],
            gsem=[pltpu.SemaphoreType.DMA for _ in range(2)],
            osem=[pltpu.SemaphoreType.DMA for _ in range(2)],
        ),
        compiler_params=pltpu.CompilerParams(
            use_tc_tiling_on_sc=False, needs_layout_passes=False),
    )
    def k(tok_hbm, seg_hbm, table_hbm, comb_hbm, out_hbm,
          comb_v, tok_v, seg_v, cidx_v, rows_v, tsem, gsem, osem):
        wid = lax.axis_index("s") * NC + lax.axis_index("c")
        wbase = wid * rows_per_worker

        pltpu.sync_copy(comb_hbm, comb_v)

        def start_idx(kk, b):
            base = wbase + kk * CH
            pltpu.async_copy(tok_hbm.at[pl.ds(base, CH)], tok_v[b], tsem[b])
            pltpu.async_copy(seg_hbm.at[pl.ds(base, CH)], seg_v[b], tsem[b])

        start_idx(0, 0)
        start_idx(1, 1)

        def run_chunk(j, kk, b):
            base = wbase + kk * CH
            @pl.when(j >= 1)
            def _():
                pltpu.make_async_copy(
                    rows_v[b], out_hbm.at[pl.ds(base, CH)], osem[b]).wait()

            pltpu.make_async_copy(
                tok_hbm.at[pl.ds(base, CH)], tok_v[b], tsem[b]).wait()
            pltpu.make_async_copy(
                seg_hbm.at[pl.ds(base, CH)], seg_v[b], tsem[b]).wait()
            for g in range(CH // LANES):
                s16 = seg_v[b][pl.ds(g * LANES, LANES)]
                flat = base + g * LANES + lax.iota(jnp.int32, LANES)
                cidx_v[b][pl.ds(g * LANES, LANES)] = (
                    s16 * n_pos + lax.rem(flat, n_pos))
            gd = pltpu.async_copy(table_hbm.at[tok_v[b]], rows_v[b], gsem[b])

            @pl.when(j <= (n_chunks // 2) - 2)
            def _():
                start_idx(kk + 2, b)

            gd.wait()

            def add_group(g, c):
                r_vec = g * LANES + lax.iota(jnp.int32, LANES)
                ci = cidx_v[b][pl.ds(g * LANES, LANES)]
                for dd in range(d):
                    d_vec = jnp.full((LANES,), dd, jnp.int32)
                    t = plsc.load_gather(rows_v[b], [r_vec, d_vec])
                    a = plsc.load_gather(comb_v, [ci, d_vec])
                    plsc.store_scatter(rows_v[b], [r_vec, d_vec], t + a)
                return c

            lax.fori_loop(0, CH // LANES, add_group, 0)
            pltpu.async_copy(rows_v[b], out_hbm.at[pl.ds(base, CH)], osem[b])

        def pair(j, carry):
            run_chunk(j, 2 * j, 0)
            run_chunk(j, 2 * j + 1, 1)
            return carry

        lax.fori_loop(0, n_chunks // 2, pair, 0)
        for b in range(2):
            last = wbase + (n_chunks - 2 + b) * CH
            pltpu.make_async_copy(
                rows_v[b], out_hbm.at[pl.ds(last, CH)], osem[b]).wait()

    return k(tok, seg, table, comb)


def kernel(tokens, segment, token_table, pos_table, sent_table):
    b, l = tokens.shape
    v, d = token_table.shape
    n_sent = sent_table.shape[0]
    tok = tokens.reshape(-1).astype(jnp.int32)
    seg = segment.reshape(-1).astype(jnp.int32)
    comb = (sent_table[:, None, :] + pos_table[None, :, :]).reshape(
        n_sent * l, d)
    out = _sc_embed(tok, seg, token_table, comb,
                    n_rows=b * l, d=d, n_pos=l)
    return out.reshape(b, l, d)


# speedup vs baseline: 3.7983x; 3.7983x over previous
import functools

import jax
import jax.numpy as jnp
from jax import lax
from jax.experimental import pallas as pl
from jax.experimental.pallas import tpu as pltpu
from jax.experimental.pallas import tpu_sc as plsc

NC = 2
NS = 16
LANES = 16
CH = 128


def _sc_embed(tok, seg, table, comb, *, n_rows, d, n_pos):
    n_workers = NC * NS
    rows_per_worker = n_rows // n_workers
    n_chunks = rows_per_worker // CH
    n_comb = comb.shape[0]
    mesh = plsc.VectorSubcoreMesh(
        core_axis_name="c", subcore_axis_name="s",
        num_cores=NC, num_subcores=NS)

    @functools.partial(
        pl.kernel,
        out_type=jax.ShapeDtypeStruct((n_rows, d), jnp.float32),
        mesh=mesh,
        scratch_types=dict(
            comb_sh=pltpu.VMEM_SHARED((n_comb, d), jnp.float32),
            tok_v=[pltpu.VMEM((CH,), jnp.int32) for _ in range(2)],
            seg_v=[pltpu.VMEM((CH,), jnp.int32) for _ in range(2)],
            cidx_v=[pltpu.VMEM((CH,), jnp.int32) for _ in range(2)],
            rows_v=[pltpu.VMEM((CH, d), jnp.float32) for _ in range(2)],
            tsem=[pltpu.SemaphoreType.DMA for _ in range(2)],
            gsem=[pltpu.SemaphoreType.DMA for _ in range(2)],
            asem=[pltpu.SemaphoreType.DMA for _ in range(2)],
            osem=[pltpu.SemaphoreType.DMA for _ in range(2)],
        ),
        compiler_params=pltpu.CompilerParams(
            use_tc_tiling_on_sc=False, needs_layout_passes=False),
    )
    def k(tok_hbm, seg_hbm, table_hbm, comb_hbm, out_hbm,
          comb_sh, tok_v, seg_v, cidx_v, rows_v, tsem, gsem, asem, osem):
        wid = lax.axis_index("s") * NC + lax.axis_index("c")
        wbase = wid * rows_per_worker

        @pl.when(lax.axis_index("s") == 0)
        def _():
            pltpu.sync_copy(comb_hbm, comb_sh)

        plsc.subcore_barrier()

        def start_idx(kk, b):
            base = wbase + kk * CH
            pltpu.async_copy(tok_hbm.at[pl.ds(base, CH)], tok_v[b], tsem[b])
            pltpu.async_copy(seg_hbm.at[pl.ds(base, CH)], seg_v[b], tsem[b])

        def wait_idx(kk, b):
            base = wbase + kk * CH
            pltpu.make_async_copy(
                tok_hbm.at[pl.ds(base, CH)], tok_v[b], tsem[b]).wait()
            pltpu.make_async_copy(
                seg_hbm.at[pl.ds(base, CH)], seg_v[b], tsem[b]).wait()

        def compute_cidx(kk, b):
            base = wbase + kk * CH
            for g in range(CH // LANES):
                s16 = seg_v[b][pl.ds(g * LANES, LANES)]
                flat = base + g * LANES + lax.iota(jnp.int32, LANES)
                cidx_v[b][pl.ds(g * LANES, LANES)] = (
                    s16 * n_pos + lax.rem(flat, n_pos))

        def start_gather(b):
            pltpu.async_copy(table_hbm.at[tok_v[b]], rows_v[b], gsem[b])

        start_idx(0, 0)
        start_idx(1, 1)
        wait_idx(0, 0)
        compute_cidx(0, 0)
        start_gather(0)

        def chunk_body(j, kk, b):
            base = wbase + kk * CH
            b1 = 1 - b
            last_pair = (n_chunks // 2) - 1

            @pl.when((j <= last_pair - 1) | (b == 0))
            def _():
                wait_idx(kk + 1, b1)
                compute_cidx(kk + 1, b1)

                @pl.when(kk >= 1)
                def _():
                    pltpu.make_async_copy(
                        rows_v[b1],
                        out_hbm.at[pl.ds(base - CH, CH)], osem[b1]).wait()

                start_gather(b1)

            pltpu.make_async_copy(
                table_hbm.at[tok_v[b]], rows_v[b], gsem[b]).wait()
            pltpu.async_copy(
                comb_sh.at[cidx_v[b]], rows_v[b], asem[b], add=True)

            @pl.when(j <= last_pair - 1)
            def _():
                start_idx(kk + 2, b)

            pltpu.make_async_copy(
                comb_sh.at[cidx_v[b]], rows_v[b], asem[b]).wait()
            pltpu.async_copy(rows_v[b], out_hbm.at[pl.ds(base, CH)], osem[b])

        def pair(j, carry):
            chunk_body(j, 2 * j, 0)
            chunk_body(j, 2 * j + 1, 1)
            return carry

        lax.fori_loop(0, n_chunks // 2, pair, 0)
        for b in range(2):
            last = wbase + (n_chunks - 2 + b) * CH
            pltpu.make_async_copy(
                rows_v[b], out_hbm.at[pl.ds(last, CH)], osem[b]).wait()

    return k(tok, seg, table, comb)


def kernel(tokens, segment, token_table, pos_table, sent_table):
    b, l = tokens.shape
    v, d = token_table.shape
    n_sent = sent_table.shape[0]
    tok = tokens.reshape(-1).astype(jnp.int32)
    seg = segment.reshape(-1).astype(jnp.int32)
    comb = (sent_table[:, None, :] + pos_table[None, :, :]).reshape(
        n_sent * l, d)
    out = _sc_embed(tok, seg, token_table, comb,
                    n_rows=b * l, d=d, n_pos=l)
    return out.reshape(b, l, d)
